# Initial kernel scaffold; baseline (speedup 1.0000x reference)
import jax
import jax.numpy as jnp
from jax.experimental import pallas as pl


def kernel(input_ids, attention_mask, emb, W):
    raise NotImplementedError("write your pallas kernel here")


# speedup vs baseline: 8.2997x; 8.2997x over previous
import functools

import jax
import jax.numpy as jnp
from jax import lax
from jax.experimental import pallas as pl
from jax.experimental.pallas import tpu as pltpu
from jax.experimental.pallas import tpu_sc as plsc

NC = 2
NS = 16
NW = NC * NS
LANES = 16

_L = 200
_D = 16
_CHUNK = 128
_SUP_IDXROWS = 25
_SUP_IDX = _SUP_IDXROWS * _CHUNK
_SUP_B = _SUP_IDX // _L


def _pool_sc(ids2d, emb, B):
    rows_per_tile = B // NW
    sup_per_tile = rows_per_tile // _SUP_B
    idxrows_per_tile = ids2d.shape[0] // NW

    mesh = plsc.VectorSubcoreMesh(
        core_axis_name="c", subcore_axis_name="s", num_cores=NC,
        num_subcores=NS)

    @functools.partial(
        pl.kernel,
        out_type=jax.ShapeDtypeStruct((B, _D), jnp.float32),
        mesh=mesh,
        scratch_types=[
            pltpu.VMEM((_SUP_IDXROWS, _CHUNK), jnp.int32),
            pltpu.VMEM((_SUP_IDX, _D), jnp.float32),
            pltpu.VMEM((rows_per_tile, _D), jnp.float32),
            pltpu.SemaphoreType.DMA,
        ],
        compiler_params=pltpu.CompilerParams(use_tc_tiling_on_sc=False),
    )
    def k(ids_hbm, emb_hbm, out_hbm, ibuf, rowsbuf, outbuf, sem):
        wid = lax.axis_index("s") * NC + lax.axis_index("c")
        idxrow0 = wid * idxrows_per_tile

        def superchunk(t, _):
            pltpu.sync_copy(
                ids_hbm.at[pl.ds(idxrow0 + t * _SUP_IDXROWS, _SUP_IDXROWS)],
                ibuf)

            def fire(c, _):
                pltpu.async_copy(
                    emb_hbm.at[ibuf.at[c]],
                    rowsbuf.at[pl.ds(c * _CHUNK, _CHUNK)],
                    sem)
                return 0
            lax.fori_loop(0, _SUP_IDXROWS, fire, 0)
            pltpu.make_async_copy(
                emb_hbm.at[pl.ds(0, _SUP_IDX)], rowsbuf, sem).wait()

            def pool_row(g, _):
                base = g * _L

                def acc_step(i, accs):
                    j = base + i * 8
                    return tuple(
                        accs[k] + rowsbuf[j + k, :] for k in range(8))

                zero = jnp.zeros((LANES,), jnp.float32)
                accs = lax.fori_loop(0, _L // 8, acc_step, (zero,) * 8)
                s4 = (accs[0] + accs[1], accs[2] + accs[3],
                      accs[4] + accs[5], accs[6] + accs[7])
                outbuf[t * _SUP_B + g, :] = (s4[0] + s4[1]) + (s4[2] + s4[3])
                return 0
            lax.fori_loop(0, _SUP_B, pool_row, 0)
            return 0

        lax.fori_loop(0, sup_per_tile, superchunk, 0)
        pltpu.sync_copy(outbuf,
                        out_hbm.at[pl.ds(wid * rows_per_tile, rows_per_tile)])

    return k(ids2d, emb)


def _project_tc(s, W):
    def body(s_ref, w_ref, o_ref):
        o_ref[...] = lax.dot_general(
            s_ref[...], w_ref[...], (((1,), (1,)), ((), ())),
            preferred_element_type=jnp.float32)

    return pl.pallas_call(
        body,
        out_shape=jax.ShapeDtypeStruct((s.shape[0], W.shape[0]), jnp.float32),
    )(s, W)


@jax.jit
def kernel(input_ids, attention_mask, emb, W):
    del attention_mask
    B, L = input_ids.shape
    ids2d = input_ids.reshape(B * L // _CHUNK, _CHUNK)
    pooled = _pool_sc(ids2d, emb, B)
    return _project_tc(pooled, W)


# speedup vs baseline: 9.0866x; 1.0948x over previous
import functools

import jax
import jax.numpy as jnp
from jax import lax
from jax.experimental import pallas as pl
from jax.experimental.pallas import tpu as pltpu
from jax.experimental.pallas import tpu_sc as plsc

NC = 2
NS = 16
NW = NC * NS
LANES = 16

_L = 200
_D = 16
_CHUNK = 128
_SUP_IDXROWS = 25
_SUP_IDX = _SUP_IDXROWS * _CHUNK
_SUP_B = _SUP_IDX // _L
_UNROLL = 25


def _pool_sc(ids2d, emb, B):
    rows_per_tile = B // NW
    sup_per_tile = rows_per_tile // _SUP_B
    idxrows_per_tile = ids2d.shape[0] // NW

    mesh = plsc.VectorSubcoreMesh(
        core_axis_name="c", subcore_axis_name="s", num_cores=NC,
        num_subcores=NS)

    @functools.partial(
        pl.kernel,
        out_type=jax.ShapeDtypeStruct((B, _D), jnp.float32),
        mesh=mesh,
        scratch_types=[
            pltpu.VMEM((_SUP_IDXROWS, _CHUNK), jnp.int32),
            pltpu.VMEM((_SUP_IDX, _D), jnp.float32),
            pltpu.VMEM((_SUP_IDX, _D), jnp.float32),
            pltpu.VMEM((rows_per_tile, _D), jnp.float32),
            pltpu.SemaphoreType.DMA,
            pltpu.SemaphoreType.DMA,
        ],
        compiler_params=pltpu.CompilerParams(use_tc_tiling_on_sc=False),
    )
    def k(ids_hbm, emb_hbm, out_hbm, ibuf, rows0, rows1, outbuf, sg0, sg1):
        wid = lax.axis_index("s") * NC + lax.axis_index("c")
        idxrow0 = wid * idxrows_per_tile
        rbufs = (rows0, rows1)
        sgs = (sg0, sg1)

        def idx_src(t):
            return ids_hbm.at[
                pl.ds(idxrow0 + t * _SUP_IDXROWS, _SUP_IDXROWS)]

        def fire_gathers(ib, rb, sg):
            def fire(c, _):
                pltpu.async_copy(emb_hbm.at[ib.at[c]],
                                 rb.at[pl.ds(c * _CHUNK, _CHUNK)], sg)
                return 0
            lax.fori_loop(0, _SUP_IDXROWS, fire, 0)

        def drain_gathers(rb, sg):
            pltpu.make_async_copy(
                emb_hbm.at[pl.ds(0, _SUP_IDX)], rb, sg).wait()

        def accumulate(rb, t):
            def pool_row(g, _):
                base = g * _L

                def acc_step(i, accs):
                    j = base + i * _UNROLL
                    accs = list(accs)
                    for k in range(_UNROLL):
                        accs[k % 8] = accs[k % 8] + rb[j + k, :]
                    return tuple(accs)

                zero = jnp.zeros((LANES,), jnp.float32)
                accs = lax.fori_loop(0, _L // _UNROLL, acc_step, (zero,) * 8)
                s4 = (accs[0] + accs[1], accs[2] + accs[3],
                      accs[4] + accs[5], accs[6] + accs[7])
                outbuf[t * _SUP_B + g, :] = (s4[0] + s4[1]) + (s4[2] + s4[3])
                return 0
            lax.fori_loop(0, _SUP_B, pool_row, 0)

        pltpu.sync_copy(idx_src(0), ibuf)
        fire_gathers(ibuf, rows0, sg0)

        def pair(t2, _):
            for p in (0, 1):
                t = 2 * t2 + p
                drain_gathers(rbufs[p], sgs[p])
                pltpu.sync_copy(idx_src(t + 1), ibuf)
                fire_gathers(ibuf, rbufs[1 - p], sgs[1 - p])
                accumulate(rbufs[p], t)
            return 0

        lax.fori_loop(0, sup_per_tile // 2 - 1, pair, 0)
        t_last = sup_per_tile - 2
        drain_gathers(rows0, sg0)
        pltpu.sync_copy(idx_src(t_last + 1), ibuf)
        fire_gathers(ibuf, rows1, sg1)
        accumulate(rows0, t_last)
        drain_gathers(rows1, sg1)
        accumulate(rows1, t_last + 1)
        pltpu.sync_copy(outbuf,
                        out_hbm.at[pl.ds(wid * rows_per_tile, rows_per_tile)])

    return k(ids2d, emb)


def _project_tc(s, W):
    def body(s_ref, w_ref, o_ref):
        o_ref[...] = lax.dot_general(
            s_ref[...], w_ref[...], (((1,), (1,)), ((), ())),
            preferred_element_type=jnp.float32)

    return pl.pallas_call(
        body,
        out_shape=jax.ShapeDtypeStruct((s.shape[0], W.shape[0]), jnp.float32),
    )(s, W)


@jax.jit
def kernel(input_ids, attention_mask, emb, W):
    del attention_mask
    B, L = input_ids.shape
    ids2d = input_ids.reshape(B * L // _CHUNK, _CHUNK)
    pooled = _pool_sc(ids2d, emb, B)
    return _project_tc(pooled, W)
